# baseline (speedup 1.0000x reference)
import functools

import jax
import jax.numpy as jnp
from jax import lax
from jax.experimental import pallas as pl
from jax.experimental.pallas import tpu as pltpu
from jax.experimental.pallas import tpu_sc as plsc

_D = 64
_DP = 128
_B = 4096
_L = 50
_NW = 32
_PPW = _B // _NW
_CP = 2
_J = _PPW // _CP
_NBUF = 2
_G = _J // _NBUF


def _make_sc_embed():
  mesh = plsc.VectorSubcoreMesh(core_axis_name="c", subcore_axis_name="s")

  @functools.partial(
      pl.kernel,
      mesh=mesh,
      compiler_params=pltpu.CompilerParams(use_tc_tiling_on_sc=True),
      out_type=jax.ShapeDtypeStruct((_B, _L, _D), jnp.float32),
      scratch_types=[
          pltpu.VMEM((_PPW, _L), jnp.int32),
          pltpu.VMEM((_L, _D), jnp.float32),
          pltpu.VMEM((_NBUF, _CP, _L, _DP), jnp.float32),
          pltpu.VMEM((_NBUF, _CP, _L, _D), jnp.float32),
          pltpu.SemaphoreType.DMA,
          pltpu.SemaphoreType.DMA,
          pltpu.SemaphoreType.DMA,
          pltpu.SemaphoreType.DMA,
      ],
  )
  def embed(idx_hbm, table_hbm, pos_hbm, out_hbm, idx_v, pos_v, gbuf, obuf,
            gsem0, gsem1, ssem0, ssem1):
    gsems = (gsem0, gsem1)
    ssems = (ssem0, ssem1)
    cid = lax.axis_index("c")
    sid = lax.axis_index("s")
    wid = sid * 2 + cid
    pbase = wid * _PPW
    pltpu.sync_copy(idx_hbm.at[pl.ds(pbase, _PPW)], idx_v)
    pltpu.sync_copy(pos_hbm, pos_v)

    def gather_desc(local_p, slot, q, sem):
      return pltpu.make_async_copy(
          table_hbm.at[idx_v.at[local_p]], gbuf.at[slot, q], sem)

    def store_desc(slot, local_p, sem):
      return pltpu.make_async_copy(
          obuf.at[slot], out_hbm.at[pl.ds(pbase + local_p, _CP)], sem)

    for b in range(_NBUF):
      for q in range(_CP):
        gather_desc(b * _CP + q, b, q, gsems[b]).start()

    def body(gh, carry):
      pvs = [[pos_v[r, pl.ds(16 * s, 16)] for s in range(_D // 16)]
             for r in range(_L)]
      for rep in range(2):
       g = gh * 2 + rep
       for b in range(_NBUF):
        lp = (g * _NBUF + b) * _CP
        for q in range(_CP):
          gather_desc(lp + q, b, q, gsems[b]).wait()

        @pl.when(g > 0)
        def _():
          store_desc(b, lp - _NBUF * _CP, ssems[b]).wait()

        for r in range(_L):
          for s in range(_D // 16):
            sl = pl.ds(16 * s, 16)
            for q in range(_CP):
              obuf[b, q, r, sl] = gbuf[b, q, r, sl] + pvs[r][s]

        @pl.when(g < _G - 1)
        def _():
          nxt = lp + _NBUF * _CP
          for q in range(_CP):
            gather_desc(nxt + q, b, q, gsems[b]).start()

        store_desc(b, lp, ssems[b]).start()
      return carry

    lax.fori_loop(0, _G // 2, body, 0)

    for b in range(_NBUF):
      lp = ((_G - 1) * _NBUF + b) * _CP
      store_desc(b, lp, ssems[b]).wait()

  return embed


_sc_embed = _make_sc_embed()


def kernel(phrase, phrase_emb_weight, pos_emb_weight):
  table_p = jnp.pad(phrase_emb_weight, ((0, 0), (0, _DP - _D)))
  return _sc_embed(phrase.astype(jnp.int32), table_p, pos_emb_weight)


# speedup vs baseline: 1.0767x; 1.0767x over previous
import functools

import jax
import jax.numpy as jnp
from jax import lax
from jax.experimental import pallas as pl
from jax.experimental.pallas import tpu as pltpu
from jax.experimental.pallas import tpu_sc as plsc

_D = 64
_DP = 128
_B = 4096
_L = 50
_NW = 32
_PPW = _B // _NW
_CP = 2
_J = _PPW // _CP
_NBUF = 2
_G = _J // _NBUF


def _make_sc_embed():
  mesh = plsc.VectorSubcoreMesh(core_axis_name="c", subcore_axis_name="s")

  @functools.partial(
      pl.kernel,
      mesh=mesh,
      compiler_params=pltpu.CompilerParams(use_tc_tiling_on_sc=True),
      out_type=jax.ShapeDtypeStruct((_B, _L, _D), jnp.float32),
      scratch_types=[
          pltpu.VMEM((_PPW, _L), jnp.int32),
          pltpu.VMEM((_L, _D), jnp.float32),
          pltpu.VMEM((_NBUF, _CP, _L, _DP), jnp.float32),
          pltpu.VMEM((_NBUF, _CP, _L, _D), jnp.float32),
          pltpu.SemaphoreType.DMA,
          pltpu.SemaphoreType.DMA,
          pltpu.SemaphoreType.DMA,
          pltpu.SemaphoreType.DMA,
      ],
  )
  def embed(idx_hbm, table_hbm, pos_hbm, out_hbm, idx_v, pos_v, gbuf, obuf,
            gsem0, gsem1, ssem0, ssem1):
    gsems = (gsem0, gsem1)
    ssems = (ssem0, ssem1)
    cid = lax.axis_index("c")
    sid = lax.axis_index("s")
    wid = sid * 2 + cid
    pbase = wid * _PPW
    pltpu.sync_copy(idx_hbm.at[pl.ds(pbase, _PPW)], idx_v)
    pltpu.sync_copy(pos_hbm, pos_v)

    def gather_desc(local_p, slot, q, sem):
      return pltpu.make_async_copy(
          table_hbm.at[idx_v.at[local_p]], gbuf.at[slot, q], sem)

    def store_desc(slot, local_p, sem):
      return pltpu.make_async_copy(
          obuf.at[slot], out_hbm.at[pl.ds(pbase + local_p, _CP)], sem)

    for b in range(_NBUF):
      for q in range(_CP):
        gather_desc(b * _CP + q, b, q, gsems[b]).start()

    def body(g, carry):
      pvs = [[pos_v[r, pl.ds(16 * s, 16)] for s in range(_D // 16)]
             for r in range(_L)]
      for b in range(_NBUF):
        lp = (g * _NBUF + b) * _CP
        for q in range(_CP):
          gather_desc(lp + q, b, q, gsems[b]).wait()

        @pl.when(g > 0)
        def _():
          store_desc(b, lp - _NBUF * _CP, ssems[b]).wait()

        for r in range(_L):
          for s in range(_D // 16):
            sl = pl.ds(16 * s, 16)
            for q in range(_CP):
              obuf[b, q, r, sl] = gbuf[b, q, r, sl] + pvs[r][s]

        @pl.when(g < _G - 1)
        def _():
          nxt = lp + _NBUF * _CP
          for q in range(_CP):
            gather_desc(nxt + q, b, q, gsems[b]).start()

        store_desc(b, lp, ssems[b]).start()
      return carry

    lax.fori_loop(0, _G, body, 0)

    for b in range(_NBUF):
      lp = ((_G - 1) * _NBUF + b) * _CP
      store_desc(b, lp, ssems[b]).wait()

  return embed


_sc_embed = _make_sc_embed()


def kernel(phrase, phrase_emb_weight, pos_emb_weight):
  table_p = jnp.pad(phrase_emb_weight, ((0, 0), (0, _DP - _D)))
  return _sc_embed(phrase.astype(jnp.int32), table_p, pos_emb_weight)
